# Initial kernel scaffold; baseline (speedup 1.0000x reference)
import functools

import numpy as np
import jax
import jax.numpy as jnp
from jax import lax
from jax.experimental import pallas as pl
from jax.experimental.pallas import tpu as pltpu
from jax.experimental.pallas import tpu_sc as plsc

N_CAT = 8
VOCAB = 100000
EMB = 32
B = 1024
L = 200
D_MODEL = N_CAT * EMB
LANES = 16

NC, NS = 2, 16
NW = NC * NS
BPW = B // NW
ROWS = N_CAT * L
HALF = ROWS // 2
CHUNK = 80
N_CHUNKS = HALF // CHUNK


def _pe_rows() -> np.ndarray:
    position = np.arange(L, dtype=np.float32)[:, None]
    div_term = np.exp(
        np.arange(0, D_MODEL, 2, dtype=np.float32) * (-np.log(10000.0) / D_MODEL)
    )
    pe = np.zeros((L, D_MODEL), dtype=np.float32)
    pe[:, 0::2] = np.sin(position * div_term)
    pe[:, 1::2] = np.cos(position * div_term)
    return pe.reshape(ROWS, EMB)


_PE_ROWS = _pe_rows()

_mesh = plsc.VectorSubcoreMesh(core_axis_name="c", subcore_axis_name="s")


@functools.partial(
    pl.kernel,
    out_type=jax.ShapeDtypeStruct((B * ROWS, EMB), jnp.float32),
    mesh=_mesh,
    scratch_types=[
        pltpu.VMEM((ROWS, EMB), jnp.float32),
        pltpu.VMEM((ROWS,), jnp.int32),
        pltpu.VMEM((ROWS,), jnp.int32),
        pltpu.VMEM((HALF, EMB), jnp.float32),
        pltpu.SemaphoreType.DMA,
    ],
)
def _emb_lookup(table_hbm, idx_hbm, pe_hbm, out_hbm, pe_v, raw_v, rowid_v, buf, sem_g):
    wid = lax.axis_index("s") * NC + lax.axis_index("c")
    pltpu.sync_copy(pe_hbm, pe_v)
    iota = lax.broadcasted_iota(jnp.int32, (LANES,), 0)

    def per_batch(bi, _):
        b = wid * BPW + bi
        pltpu.sync_copy(idx_hbm.at[b], raw_v)

        def rowid_step(j, _):
            t = j * LANES + iota
            l = lax.shift_right_logical(t, 3)
            c = lax.bitwise_and(t, 7)
            src = c * L + l
            v = plsc.load_gather(raw_v, [src])
            rowid_v[pl.ds(j * LANES, LANES)] = v + c * VOCAB
            return 0

        lax.fori_loop(0, ROWS // LANES, rowid_step, 0)

        def per_half(h, _):
            base = h * HALF
            descs = [
                pltpu.async_copy(
                    table_hbm.at[rowid_v.at[pl.ds(base + k * CHUNK, CHUNK)]],
                    buf.at[pl.ds(k * CHUNK, CHUNK)],
                    sem_g,
                )
                for k in range(N_CHUNKS)
            ]
            for d in descs:
                d.wait()

            def add_pe(r, _):
                buf[r, pl.ds(0, LANES)] += pe_v[base + r, pl.ds(0, LANES)]
                buf[r, pl.ds(LANES, LANES)] += pe_v[base + r, pl.ds(LANES, LANES)]
                return 0

            lax.fori_loop(0, HALF, add_pe, 0)
            row0 = (b * 2 + h) * HALF
            pltpu.sync_copy(buf, out_hbm.at[pl.ds(row0, HALF)])
            return 0

        lax.fori_loop(0, 2, per_half, 0)
        return 0

    lax.fori_loop(0, BPW, per_batch, 0)


def kernel(tables, categorical_attrs):
    table = tables.reshape(N_CAT * VOCAB, EMB)
    idx = categorical_attrs.astype(jnp.int32).reshape(B, N_CAT * L)
    pe = jnp.asarray(_PE_ROWS)
    out = _emb_lookup(table, idx, pe)
    return out.reshape(B, L, D_MODEL)


# speedup vs baseline: 6.5880x; 6.5880x over previous
import functools

import numpy as np
import jax
import jax.numpy as jnp
from jax import lax
from jax.experimental import pallas as pl
from jax.experimental.pallas import tpu as pltpu
from jax.experimental.pallas import tpu_sc as plsc

N_CAT = 8
VOCAB = 100000
EMB = 32
B = 1024
L = 200
D_MODEL = N_CAT * EMB
LANES = 16

NC, NS = 2, 16
NW = NC * NS
BPW = B // NW
ROWS = N_CAT * L
HALF = ROWS // 2
CHUNK = 80
N_CHUNKS = HALF // CHUNK


def _pe_rows() -> np.ndarray:
    position = np.arange(L, dtype=np.float32)[:, None]
    div_term = np.exp(
        np.arange(0, D_MODEL, 2, dtype=np.float32) * (-np.log(10000.0) / D_MODEL)
    )
    pe = np.zeros((L, D_MODEL), dtype=np.float32)
    pe[:, 0::2] = np.sin(position * div_term)
    pe[:, 1::2] = np.cos(position * div_term)
    return pe.reshape(ROWS, EMB)


_PE_ROWS = _pe_rows()

_mesh = plsc.VectorSubcoreMesh(core_axis_name="c", subcore_axis_name="s")


@functools.partial(
    pl.kernel,
    out_type=jax.ShapeDtypeStruct((B * ROWS, EMB), jnp.float32),
    mesh=_mesh,
    scratch_types=[
        pltpu.VMEM((ROWS, EMB), jnp.float32),
        pltpu.VMEM((ROWS,), jnp.int32),
        pltpu.VMEM((ROWS,), jnp.int32),
        pltpu.VMEM((HALF, EMB), jnp.float32),
        pltpu.SemaphoreType.DMA,
    ],
    compiler_params=pltpu.CompilerParams(
        needs_layout_passes=False, use_tc_tiling_on_sc=False
    ),
)
def _emb_lookup(table_hbm, idx_hbm, pe_hbm, out_hbm, pe_v, raw_v, rowid_v, buf, sem_g):
    wid = lax.axis_index("s") * NC + lax.axis_index("c")
    pltpu.sync_copy(pe_hbm, pe_v)
    iota = lax.broadcasted_iota(jnp.int32, (LANES,), 0)

    def per_batch(bi, _):
        b = wid * BPW + bi
        pltpu.sync_copy(idx_hbm.at[b], raw_v)

        def rowid_step(j, _):
            t = j * LANES + iota
            l = lax.shift_right_logical(t, 3)
            c = lax.bitwise_and(t, 7)
            src = c * L + l
            v = plsc.load_gather(raw_v, [src])
            rowid_v[pl.ds(j * LANES, LANES)] = v + c * VOCAB
            return 0

        lax.fori_loop(0, ROWS // LANES, rowid_step, 0)

        def per_half(h, _):
            base = h * HALF
            descs = [
                pltpu.async_copy(
                    table_hbm.at[rowid_v.at[pl.ds(base + k * CHUNK, CHUNK)]],
                    buf.at[pl.ds(k * CHUNK, CHUNK)],
                    sem_g,
                )
                for k in range(N_CHUNKS)
            ]
            for d in descs:
                d.wait()

            def add_pe(r, _):
                buf[r, pl.ds(0, LANES)] += pe_v[base + r, pl.ds(0, LANES)]
                buf[r, pl.ds(LANES, LANES)] += pe_v[base + r, pl.ds(LANES, LANES)]
                return 0

            lax.fori_loop(0, HALF, add_pe, 0)
            row0 = (b * 2 + h) * HALF
            pltpu.sync_copy(buf, out_hbm.at[pl.ds(row0, HALF)])
            return 0

        lax.fori_loop(0, 2, per_half, 0)
        return 0

    lax.fori_loop(0, BPW, per_batch, 0)


def kernel(tables, categorical_attrs):
    table = tables.reshape(N_CAT * VOCAB, EMB)
    idx = categorical_attrs.astype(jnp.int32).reshape(B, N_CAT * L)
    pe = jnp.asarray(_PE_ROWS)
    out = _emb_lookup(table, idx, pe)
    return out.reshape(B, L, D_MODEL)


# speedup vs baseline: 8.2788x; 1.2566x over previous
import functools

import numpy as np
import jax
import jax.numpy as jnp
from jax import lax
from jax.experimental import pallas as pl
from jax.experimental.pallas import tpu as pltpu
from jax.experimental.pallas import tpu_sc as plsc

N_CAT = 8
VOCAB = 100000
EMB = 32
B = 1024
L = 200
D_MODEL = N_CAT * EMB
LANES = 16

NC, NS = 2, 16
NW = NC * NS
BPW = B // NW
ROWS = N_CAT * L
NQ = 4
QROWS = ROWS // NQ
CHUNK = 80
N_CHUNKS = QROWS // CHUNK


def _pe_rows() -> np.ndarray:
    position = np.arange(L, dtype=np.float32)[:, None]
    div_term = np.exp(
        np.arange(0, D_MODEL, 2, dtype=np.float32) * (-np.log(10000.0) / D_MODEL)
    )
    pe = np.zeros((L, D_MODEL), dtype=np.float32)
    pe[:, 0::2] = np.sin(position * div_term)
    pe[:, 1::2] = np.cos(position * div_term)
    return pe.reshape(ROWS, EMB)


_PE_ROWS = _pe_rows()

_mesh = plsc.VectorSubcoreMesh(core_axis_name="c", subcore_axis_name="s")


@functools.partial(
    pl.kernel,
    out_type=jax.ShapeDtypeStruct((B * ROWS, EMB), jnp.float32),
    mesh=_mesh,
    scratch_types=[
        pltpu.VMEM((ROWS, EMB), jnp.float32),
        pltpu.VMEM((ROWS,), jnp.int32),
        pltpu.VMEM((ROWS,), jnp.int32),
        [pltpu.VMEM((QROWS, EMB), jnp.float32) for _ in range(NQ)],
        [pltpu.SemaphoreType.DMA for _ in range(NQ)],
        [pltpu.SemaphoreType.DMA for _ in range(NQ)],
    ],
    compiler_params=pltpu.CompilerParams(
        needs_layout_passes=False, use_tc_tiling_on_sc=False
    ),
)
def _emb_lookup(table_hbm, idx_hbm, pe_hbm, out_hbm, pe_v, raw_v, rowid_v, bufs, gsems, wsems):
    wid = lax.axis_index("s") * NC + lax.axis_index("c")
    pltpu.sync_copy(pe_hbm, pe_v)
    iota = lax.broadcasted_iota(jnp.int32, (LANES,), 0)

    def per_batch(bi, _):
        b = wid * BPW + bi
        pltpu.sync_copy(idx_hbm.at[b], raw_v)

        @plsc.parallel_loop(0, ROWS // LANES, unroll=4)
        def _rowid_step(j):
            t = j * LANES + iota
            l = lax.shift_right_logical(t, 3)
            c = lax.bitwise_and(t, 7)
            v = plsc.load_gather(raw_v, [c * L + l])
            rowid_v[pl.ds(j * LANES, LANES)] = v + c * VOCAB

        for q in range(NQ):
            @pl.when(bi > 0)
            def _drain(q=q):
                row0 = ((b - 1) * NQ + q) * QROWS
                pltpu.make_async_copy(
                    bufs[q], out_hbm.at[pl.ds(row0, QROWS)], wsems[q]
                ).wait()

            for k in range(N_CHUNKS):
                pltpu.async_copy(
                    table_hbm.at[rowid_v.at[pl.ds(q * QROWS + k * CHUNK, CHUNK)]],
                    bufs[q].at[pl.ds(k * CHUNK, CHUNK)],
                    gsems[q],
                )

        for q in range(NQ):
            for k in range(N_CHUNKS):
                pltpu.make_async_copy(
                    table_hbm.at[rowid_v.at[pl.ds(q * QROWS + k * CHUNK, CHUNK)]],
                    bufs[q].at[pl.ds(k * CHUNK, CHUNK)],
                    gsems[q],
                ).wait()

            @plsc.parallel_loop(0, QROWS, unroll=4)
            def _add_pe(r, q=q):
                bufs[q][r, pl.ds(0, LANES)] += pe_v[q * QROWS + r, pl.ds(0, LANES)]
                bufs[q][r, pl.ds(LANES, LANES)] += pe_v[q * QROWS + r, pl.ds(LANES, LANES)]

            row0 = (b * NQ + q) * QROWS
            pltpu.async_copy(bufs[q], out_hbm.at[pl.ds(row0, QROWS)], wsems[q])
        return 0

    lax.fori_loop(0, BPW, per_batch, 0)

    b_last = wid * BPW + BPW - 1
    for q in range(NQ):
        row0 = (b_last * NQ + q) * QROWS
        pltpu.make_async_copy(
            bufs[q], out_hbm.at[pl.ds(row0, QROWS)], wsems[q]
        ).wait()


def kernel(tables, categorical_attrs):
    table = tables.reshape(N_CAT * VOCAB, EMB)
    idx = categorical_attrs.astype(jnp.int32).reshape(B, N_CAT * L)
    pe = jnp.asarray(_PE_ROWS)
    out = _emb_lookup(table, idx, pe)
    return out.reshape(B, L, D_MODEL)
